# baseline (speedup 1.0000x reference)
import functools

import jax
import jax.numpy as jnp
from jax import lax
from jax.experimental import pallas as pl
from jax.experimental.pallas import tpu as pltpu
from jax.experimental.pallas import tpu_sc as plsc

_NC = 2
_NS = 16
_NW = _NC * _NS
_LANES = 16
_NBUF = 4


def _add_pos(buf, posv, L, D):
    per_row = D // _LANES

    @plsc.parallel_loop(0, L * per_row, unroll=4)
    def _(i):
        l = i // per_row
        d0 = (i % per_row) * _LANES
        pv = posv[l, pl.ds(d0, _LANES)]
        plsc.addupdate(buf.at[0, l, pl.ds(d0, _LANES)], pv)


def _sc_body(x_hbm, pos_hbm, out_hbm, posv, bufs, isems, osems, L, D, nch):
    wid = lax.axis_index("s") * _NC + lax.axis_index("c")
    base = wid * nch

    pltpu.sync_copy(pos_hbm, posv)

    def start_in(c, cur):
        pltpu.async_copy(x_hbm.at[pl.ds(base + c, 1)], bufs[cur], isems[cur])

    def wait_in(c, cur):
        pltpu.make_async_copy(
            x_hbm.at[pl.ds(base + c, 1)], bufs[cur], isems[cur]
        ).wait()

    def start_out(c, cur):
        pltpu.async_copy(bufs[cur], out_hbm.at[pl.ds(base + c, 1)], osems[cur])

    def wait_out(c, cur):
        pltpu.make_async_copy(
            bufs[cur], out_hbm.at[pl.ds(base + c, 1)], osems[cur]
        ).wait()

    def process(c, cur, wait_prev, prefetch):
        wait_in(c, cur)
        _add_pos(bufs[cur], posv, L, D)
        start_out(c, cur)
        if wait_prev:
            wait_out(c - 2, (cur + 2) % _NBUF)
        if prefetch:
            start_in(c + 2, (cur + 2) % _NBUF)

    start_in(0, 0)
    start_in(1, 1)
    for cur in range(_NBUF):
        process(cur, cur, wait_prev=cur >= 2, prefetch=True)

    def outer(k, _):
        c0 = _NBUF * k
        for cur in range(_NBUF):
            process(c0 + cur, cur, wait_prev=True, prefetch=True)
        return 0

    lax.fori_loop(1, nch // _NBUF - 1, outer, 0)

    c0 = nch - _NBUF
    for cur in range(_NBUF):
        process(c0 + cur, cur, wait_prev=True, prefetch=cur < 2)

    for cur in range(2):
        wait_out(nch - 2 + cur, (cur + 2) % _NBUF)


def kernel(x, pos_table):
    B, L, D = x.shape
    nch = B // _NW
    mesh = plsc.VectorSubcoreMesh(core_axis_name="c", subcore_axis_name="s")
    body = functools.partial(_sc_body, L=L, D=D, nch=nch)
    run = pl.kernel(
        body,
        out_type=jax.ShapeDtypeStruct((B, L, D), x.dtype),
        mesh=mesh,
        compiler_params=pltpu.CompilerParams(use_tc_tiling_on_sc=True),
        scratch_types=[
            pltpu.VMEM((L, D), x.dtype),
            [pltpu.VMEM((1, L, D), x.dtype) for _ in range(_NBUF)],
            [pltpu.SemaphoreType.DMA for _ in range(_NBUF)],
            [pltpu.SemaphoreType.DMA for _ in range(_NBUF)],
        ],
    )
    return run(x, pos_table)


# speedup vs baseline: 1.0010x; 1.0010x over previous
import functools

import jax
import jax.numpy as jnp
from jax import lax
from jax.experimental import pallas as pl
from jax.experimental.pallas import tpu as pltpu
from jax.experimental.pallas import tpu_sc as plsc

_NC = 2
_NS = 16
_NW = _NC * _NS
_LANES = 16
_NBUF = 4


def _add_pos(buf, posv, L, D):
    per_row = D // _LANES

    @plsc.parallel_loop(0, (L // 8) * per_row, unroll=2)
    def _(i):
        t = i // per_row
        d0 = (i % per_row) * _LANES
        for r in range(8):
            l = t * 8 + r
            pv = posv[l, pl.ds(d0, _LANES)]
            plsc.addupdate(buf.at[0, l, pl.ds(d0, _LANES)], pv)


def _sc_body(x_hbm, pos_hbm, out_hbm, posv, bufs, isems, osems, L, D, nch):
    wid = lax.axis_index("s") * _NC + lax.axis_index("c")
    base = wid * nch

    pltpu.sync_copy(pos_hbm, posv)

    def start_in(c, cur):
        pltpu.async_copy(x_hbm.at[pl.ds(base + c, 1)], bufs[cur], isems[cur])

    def wait_in(c, cur):
        pltpu.make_async_copy(
            x_hbm.at[pl.ds(base + c, 1)], bufs[cur], isems[cur]
        ).wait()

    def start_out(c, cur):
        pltpu.async_copy(bufs[cur], out_hbm.at[pl.ds(base + c, 1)], osems[cur])

    def wait_out(c, cur):
        pltpu.make_async_copy(
            bufs[cur], out_hbm.at[pl.ds(base + c, 1)], osems[cur]
        ).wait()

    def process(c, cur, wait_prev, prefetch):
        wait_in(c, cur)
        _add_pos(bufs[cur], posv, L, D)
        start_out(c, cur)
        if wait_prev:
            wait_out(c - 2, (cur + 2) % _NBUF)
        if prefetch:
            start_in(c + 2, (cur + 2) % _NBUF)

    start_in(0, 0)
    start_in(1, 1)
    for cur in range(_NBUF):
        process(cur, cur, wait_prev=cur >= 2, prefetch=True)

    def outer(k, _):
        c0 = _NBUF * k
        for cur in range(_NBUF):
            process(c0 + cur, cur, wait_prev=True, prefetch=True)
        return 0

    lax.fori_loop(1, nch // _NBUF - 1, outer, 0)

    c0 = nch - _NBUF
    for cur in range(_NBUF):
        process(c0 + cur, cur, wait_prev=True, prefetch=cur < 2)

    for cur in range(2):
        wait_out(nch - 2 + cur, (cur + 2) % _NBUF)


def kernel(x, pos_table):
    B, L, D = x.shape
    nch = B // _NW
    mesh = plsc.VectorSubcoreMesh(core_axis_name="c", subcore_axis_name="s")
    body = functools.partial(_sc_body, L=L, D=D, nch=nch)
    run = pl.kernel(
        body,
        out_type=jax.ShapeDtypeStruct((B, L, D), x.dtype),
        mesh=mesh,
        compiler_params=pltpu.CompilerParams(use_tc_tiling_on_sc=True),
        scratch_types=[
            pltpu.VMEM((L, D), x.dtype),
            [pltpu.VMEM((1, L, D), x.dtype) for _ in range(_NBUF)],
            [pltpu.SemaphoreType.DMA for _ in range(_NBUF)],
            [pltpu.SemaphoreType.DMA for _ in range(_NBUF)],
        ],
    )
    return run(x, pos_table)


# speedup vs baseline: 1.6566x; 1.6549x over previous
import functools

import jax
import jax.numpy as jnp
from jax import lax
from jax.experimental import pallas as pl
from jax.experimental.pallas import tpu as pltpu
from jax.experimental.pallas import tpu_sc as plsc

_NC = 2
_NS = 16
_NW = _NC * _NS
_LANES = 16
_ROWS = 8
_COLS = 3200
_NBUF = 4


def _add_pos(buf, posv, col0):
    @plsc.parallel_loop(0, (_COLS // 128) * 8, unroll=2)
    def _(i):
        c = (i // 8) * 128 + (i % 8) * _LANES
        pv = posv[pl.ds(col0 + c, _LANES)]
        for r in range(_ROWS):
            plsc.addupdate(buf.at[r, pl.ds(c, _LANES)], pv)


def _sc_body(x_hbm, pos_hbm, out_hbm, posv, bufs, isems, osems, n, nch):
    wid = lax.axis_index("s") * _NC + lax.axis_index("c")
    panels = n // _COLS

    def chunk_slices(c):
        g = c // panels
        p = c % panels
        rows = pl.ds((wid * (nch // panels) + g) * _ROWS, _ROWS)
        cols = pl.ds(p * _COLS, _COLS)
        return rows, cols, p * _COLS

    pltpu.sync_copy(pos_hbm, posv)

    def start_in(c, cur):
        rows, cols, _ = chunk_slices(c)
        pltpu.async_copy(x_hbm.at[rows, cols], bufs[cur], isems[cur])

    def wait_in(c, cur):
        rows, cols, _ = chunk_slices(c)
        pltpu.make_async_copy(x_hbm.at[rows, cols], bufs[cur], isems[cur]).wait()

    def start_out(c, cur):
        rows, cols, _ = chunk_slices(c)
        pltpu.async_copy(bufs[cur], out_hbm.at[rows, cols], osems[cur])

    def wait_out(c, cur):
        rows, cols, _ = chunk_slices(c)
        pltpu.make_async_copy(bufs[cur], out_hbm.at[rows, cols], osems[cur]).wait()

    def process(c, cur, wait_prev, prefetch):
        wait_in(c, cur)
        _, _, col0 = chunk_slices(c)
        _add_pos(bufs[cur], posv, col0)
        start_out(c, cur)
        if wait_prev:
            wait_out(c - 2, (cur + 2) % _NBUF)
        if prefetch:
            start_in(c + 2, (cur + 2) % _NBUF)

    start_in(0, 0)
    start_in(1, 1)
    for cur in range(_NBUF):
        process(cur, cur, wait_prev=cur >= 2, prefetch=True)

    def outer(k, _):
        c0 = _NBUF * k
        for cur in range(_NBUF):
            process(c0 + cur, cur, wait_prev=True, prefetch=True)
        return 0

    lax.fori_loop(1, nch // _NBUF - 1, outer, 0)

    c0 = nch - _NBUF
    for cur in range(_NBUF):
        process(c0 + cur, cur, wait_prev=True, prefetch=cur < 2)

    for cur in range(2):
        wait_out(nch - 2 + cur, (cur + 2) % _NBUF)


def kernel(x, pos_table):
    B, L, D = x.shape
    N = L * D
    panels = N // _COLS
    nch = (B // _NW // _ROWS) * panels
    mesh = plsc.VectorSubcoreMesh(core_axis_name="c", subcore_axis_name="s")
    body = functools.partial(_sc_body, n=N, nch=nch)
    run = pl.kernel(
        body,
        out_type=jax.ShapeDtypeStruct((B, N), x.dtype),
        mesh=mesh,
        compiler_params=pltpu.CompilerParams(use_tc_tiling_on_sc=True),
        scratch_types=[
            pltpu.VMEM((N,), x.dtype),
            [pltpu.VMEM((_ROWS, _COLS), x.dtype) for _ in range(_NBUF)],
            [pltpu.SemaphoreType.DMA for _ in range(_NBUF)],
            [pltpu.SemaphoreType.DMA for _ in range(_NBUF)],
        ],
    )
    out = run(x.reshape(B, N), pos_table.reshape(N))
    return out.reshape(B, L, D)
